# baseline (speedup 1.0000x reference)
import functools

import jax
import jax.numpy as jnp
from jax import lax
from jax.experimental import pallas as pl
from jax.experimental.pallas import tpu as pltpu
from jax.experimental.pallas import tpu_sc as plsc

_SC_INFO = plsc.get_sparse_core_info()
_NC = _SC_INFO.num_cores
_NS = _SC_INFO.num_subcores
_NW = _NC * _NS
_L = _SC_INFO.num_lanes

_INV_SQRT2 = 0.7071067811865476
_ALPHA = (-2.72614225801306e-10, 2.77068142495902e-08, -2.10102402082508e-06,
          -5.69250639462346e-05, -7.34990630326855e-04, -2.95459980854025e-03,
          -1.60960333262415e-02)
_BETA = (-1.45660718464996e-05, -2.13374055278905e-04, -1.68282697438203e-03,
         -7.37332916720468e-03, -1.42647390514189e-02)


def _flat_idx16(x, n, ncols, pos):
    f32 = jnp.float32
    z = x * f32(_INV_SQRT2)
    z = jnp.minimum(jnp.maximum(z, f32(-4.0)), f32(4.0))
    z2 = z * z
    p = jnp.full((_L,), _ALPHA[0], dtype=jnp.float32)
    for c in _ALPHA[1:]:
        p = p * z2 + f32(c)
    p = p * z
    q = jnp.full((_L,), _BETA[0], dtype=jnp.float32)
    for c in _BETA[1:]:
        q = q * z2 + f32(c)
    unif = f32(0.5) + f32(0.5) * (p / q)
    idx = (unif * f32(n)).astype(jnp.int32)
    idx = jnp.minimum(jnp.maximum(idx, 0), n - 1)
    col = lax.rem(pos, ncols)
    return idx * ncols + col


def _make_sc_kernel(total, n, ncols, chunk):
    per_w = total // _NW
    n_chunks = per_w // chunk
    n_vec = chunk // _L
    mesh = plsc.VectorSubcoreMesh(core_axis_name="c", subcore_axis_name="s")

    @functools.partial(
        pl.kernel,
        mesh=mesh,
        out_type=jax.ShapeDtypeStruct((total,), jnp.float32),
        scratch_types=[
            pltpu.VMEM((chunk,), jnp.float32),
            pltpu.VMEM((chunk,), jnp.float32),
            pltpu.VMEM((chunk,), jnp.int32),
            pltpu.VMEM((chunk,), jnp.int32),
            pltpu.VMEM((chunk,), jnp.float32),
            pltpu.VMEM((chunk,), jnp.float32),
            pltpu.SemaphoreType.DMA,
            pltpu.SemaphoreType.DMA,
            pltpu.SemaphoreType.DMA,
            pltpu.SemaphoreType.DMA,
            pltpu.SemaphoreType.DMA,
            pltpu.SemaphoreType.DMA,
        ],
    )
    def sc_body(order_hbm, noise_hbm, out_hbm,
                nv0, nv1, iv0, iv1, rv0, rv1,
                ns0, ns1, gs0, gs1, os0, os1):
        wid = lax.axis_index("s") * _NC + lax.axis_index("c")
        base = wid * per_w
        nvs, ivs, rvs = (nv0, nv1), (iv0, iv1), (rv0, rv1)
        nsems, gsems, osems = (ns0, ns1), (gs0, gs1), (os0, os1)
        noise_cps = [None, None]
        gathers = [None, None]
        stores = [None, None]
        iota = lax.iota(jnp.int32, _L)

        noise_cps[0] = pltpu.async_copy(
            noise_hbm.at[pl.ds(base, chunk)], nvs[0], nsems[0])
        for k in range(n_chunks):
            s = k % 2
            o = 1 - s
            if k + 1 < n_chunks:
                noise_cps[o] = pltpu.async_copy(
                    noise_hbm.at[pl.ds(base + (k + 1) * chunk, chunk)],
                    nvs[o], nsems[o])
            noise_cps[s].wait()
            nv, iv = nvs[s], ivs[s]

            def compute(v, carry, nv=nv, iv=iv):
                voff = v * _L
                x = nv[pl.ds(voff, _L)]
                iv[pl.ds(voff, _L)] = _flat_idx16(x, n, ncols, voff + iota)
                return carry

            lax.fori_loop(0, n_vec, compute, 0, unroll=4)

            if stores[s] is not None:
                stores[s].wait()
            gathers[s] = pltpu.async_copy(
                order_hbm.at[ivs[s]], rvs[s], gsems[s])
            if gathers[o] is not None:
                gathers[o].wait()
                stores[o] = pltpu.async_copy(
                    rvs[o], out_hbm.at[pl.ds(base + (k - 1) * chunk, chunk)],
                    osems[o])
        s_last = (n_chunks - 1) % 2
        gathers[s_last].wait()
        stores[s_last] = pltpu.async_copy(
            rvs[s_last],
            out_hbm.at[pl.ds(base + (n_chunks - 1) * chunk, chunk)],
            osems[s_last])
        stores[0].wait()
        stores[1].wait()

    return sc_body


def kernel(noise, order):
    n, ncols = order.shape
    b = noise.shape[0]
    total = b * ncols
    out_flat = _make_sc_kernel(total, n, ncols, 16384)(
        order.reshape(-1), noise.reshape(-1))
    return out_flat.reshape(b, ncols)


# speedup vs baseline: 1.4008x; 1.4008x over previous
import functools

import jax
import jax.numpy as jnp
from jax import lax
from jax.experimental import pallas as pl
from jax.experimental.pallas import tpu as pltpu
from jax.experimental.pallas import tpu_sc as plsc

_SC_INFO = plsc.get_sparse_core_info()
_NC = _SC_INFO.num_cores
_NS = _SC_INFO.num_subcores
_NW = _NC * _NS


def _idx_body(n, ncols, noise_ref, out_ref):
    x = noise_ref[...]
    unif = 0.5 * (1.0 + lax.erf(x / jnp.sqrt(jnp.asarray(2.0, x.dtype))))
    idx = jnp.floor(unif * n).astype(jnp.int32)
    idx = jnp.minimum(idx, n - 1)
    col = lax.broadcasted_iota(jnp.int32, x.shape, 1)
    out_ref[...] = idx * ncols + col


def _flat_indices(noise, n, ncols):
    b = noise.shape[0]
    block_rows = 2048
    grid = (b // block_rows,)
    return pl.pallas_call(
        functools.partial(_idx_body, n, ncols),
        grid=grid,
        in_specs=[pl.BlockSpec((block_rows, ncols), lambda i: (i, 0))],
        out_specs=pl.BlockSpec((block_rows, ncols), lambda i: (i, 0)),
        out_shape=jax.ShapeDtypeStruct((b, ncols), jnp.int32),
    )(noise)


def _make_sc_gather(total, chunk):
    per_w = total // _NW
    n_chunks = per_w // chunk
    mesh = plsc.VectorSubcoreMesh(core_axis_name="c", subcore_axis_name="s")

    @functools.partial(
        pl.kernel,
        mesh=mesh,
        out_type=jax.ShapeDtypeStruct((total,), jnp.float32),
        scratch_types=[
            pltpu.VMEM((chunk,), jnp.int32),
            pltpu.VMEM((chunk,), jnp.int32),
            pltpu.VMEM((chunk,), jnp.float32),
            pltpu.VMEM((chunk,), jnp.float32),
            pltpu.SemaphoreType.DMA,
            pltpu.SemaphoreType.DMA,
            pltpu.SemaphoreType.DMA,
            pltpu.SemaphoreType.DMA,
        ],
    )
    def sc_gather(order_hbm, idx_hbm, out_hbm,
                  idx_v0, idx_v1, rows_v0, rows_v1, g0, g1, o0, o1):
        wid = lax.axis_index("s") * _NC + lax.axis_index("c")
        base = wid * per_w
        idx_bufs = (idx_v0, idx_v1)
        row_bufs = (rows_v0, rows_v1)
        gsems = (g0, g1)
        osems = (o0, o1)
        gathers = [None, None]
        stores = [None, None]
        for k in range(n_chunks):
            s = k % 2
            off = base + k * chunk
            if stores[s] is not None:
                stores[s].wait()
            pltpu.sync_copy(idx_hbm.at[pl.ds(off, chunk)], idx_bufs[s])
            gathers[s] = pltpu.async_copy(
                order_hbm.at[idx_bufs[s]], row_bufs[s], gsems[s])
            p = 1 - s
            if gathers[p] is not None:
                gathers[p].wait()
                poff = base + (k - 1) * chunk
                stores[p] = pltpu.async_copy(
                    row_bufs[p], out_hbm.at[pl.ds(poff, chunk)], osems[p])
        s_last = (n_chunks - 1) % 2
        gathers[s_last].wait()
        last_off = base + (n_chunks - 1) * chunk
        stores[s_last] = pltpu.async_copy(
            row_bufs[s_last], out_hbm.at[pl.ds(last_off, chunk)], osems[s_last])
        stores[0].wait()
        stores[1].wait()

    return sc_gather


def kernel(noise, order):
    n, ncols = order.shape
    b = noise.shape[0]
    order_flat = order.reshape(-1)
    half = b // 2
    total = half * ncols
    sc_gather = _make_sc_gather(total, 16384)
    outs = []
    for h in range(2):
        noise_h = noise[h * half:(h + 1) * half]
        flat_idx = _flat_indices(noise_h, n, ncols).reshape(-1)
        outs.append(sc_gather(order_flat, flat_idx).reshape(half, ncols))
    return jnp.concatenate(outs, axis=0)


# speedup vs baseline: 1.6206x; 1.1569x over previous
import functools

import jax
import jax.numpy as jnp
from jax import lax
from jax.experimental import pallas as pl
from jax.experimental.pallas import tpu as pltpu
from jax.experimental.pallas import tpu_sc as plsc

_SC_INFO = plsc.get_sparse_core_info()
_NC = _SC_INFO.num_cores
_NS = _SC_INFO.num_subcores
_NW = _NC * _NS


def _idx_body(n, ncols, noise_ref, out_ref):
    x = noise_ref[...]
    unif = 0.5 * (1.0 + lax.erf(x / jnp.sqrt(jnp.asarray(2.0, x.dtype))))
    idx = jnp.floor(unif * n).astype(jnp.int32)
    idx = jnp.minimum(idx, n - 1)
    col = lax.broadcasted_iota(jnp.int32, x.shape, 1)
    out_ref[...] = idx * ncols + col


def _flat_indices(noise, n, ncols):
    b = noise.shape[0]
    block_rows = 2048
    grid = (b // block_rows,)
    return pl.pallas_call(
        functools.partial(_idx_body, n, ncols),
        grid=grid,
        in_specs=[pl.BlockSpec((block_rows, ncols), lambda i: (i, 0))],
        out_specs=pl.BlockSpec((block_rows, ncols), lambda i: (i, 0)),
        out_shape=jax.ShapeDtypeStruct((b, ncols), jnp.int32),
    )(noise)


def _make_sc_gather(total, chunk):
    per_w = total // _NW
    n_chunks = per_w // chunk
    mesh = plsc.VectorSubcoreMesh(core_axis_name="c", subcore_axis_name="s")

    nbuf = 4
    depth = 2

    @functools.partial(
        pl.kernel,
        mesh=mesh,
        out_type=jax.ShapeDtypeStruct((total,), jnp.float32),
        scratch_types=(
            [pltpu.VMEM((chunk,), jnp.int32) for _ in range(nbuf)]
            + [pltpu.VMEM((chunk,), jnp.float32) for _ in range(nbuf)]
            + [pltpu.SemaphoreType.DMA for _ in range(2 * nbuf)]
        ),
    )
    def sc_gather(order_hbm, idx_hbm, out_hbm, *bufs):
        idx_bufs = bufs[:nbuf]
        row_bufs = bufs[nbuf:2 * nbuf]
        gsems = bufs[2 * nbuf:3 * nbuf]
        osems = bufs[3 * nbuf:]
        wid = lax.axis_index("s") * _NC + lax.axis_index("c")
        base = wid * per_w
        gathers = [None] * nbuf
        stores = [None] * nbuf
        for k in range(n_chunks):
            s = k % nbuf
            off = base + k * chunk
            if stores[s] is not None:
                stores[s].wait()
            pltpu.sync_copy(idx_hbm.at[pl.ds(off, chunk)], idx_bufs[s])
            gathers[s] = pltpu.async_copy(
                order_hbm.at[idx_bufs[s]], row_bufs[s], gsems[s])
            if k >= depth:
                p = (k - depth) % nbuf
                gathers[p].wait()
                poff = base + (k - depth) * chunk
                stores[p] = pltpu.async_copy(
                    row_bufs[p], out_hbm.at[pl.ds(poff, chunk)], osems[p])
        for k in range(max(n_chunks - depth, 0), n_chunks):
            p = k % nbuf
            gathers[p].wait()
            poff = base + k * chunk
            stores[p] = pltpu.async_copy(
                row_bufs[p], out_hbm.at[pl.ds(poff, chunk)], osems[p])
        for st in stores:
            if st is not None:
                st.wait()

    return sc_gather


def kernel(noise, order):
    n, ncols = order.shape
    b = noise.shape[0]
    flat_idx = _flat_indices(noise, n, ncols).reshape(-1)
    order_flat = order.reshape(-1)
    total = b * ncols
    out_flat = _make_sc_gather(total, 8192)(order_flat, flat_idx)
    return out_flat.reshape(b, ncols)
